# baseline (speedup 1.0000x reference)
import functools

import jax
import jax.numpy as jnp
import numpy as np
from jax import lax
from jax.experimental import pallas as pl
from jax.experimental.pallas import tpu as pltpu
from jax.experimental.pallas import tpu_sc as plsc

B = 8
N = 10000
D = 128
U = 128
E = 320000
M = B * N

NUM_TECS = 16
CHUNK = 128
NCHUNK = 160
NCF = NCHUNK + 4
EPAD = NUM_TECS * NCF * CHUNK
RPT = 624
TAIL = N - RPT * NUM_TECS
VSCALE = float(2 ** 24)

_BM = 2000

_PERM = np.empty(U, np.int32)
for _j in range(U // 32):
    for _t in range(16):
        _PERM[32 * _j + 2 * _t] = 32 * _j + _t
        _PERM[32 * _j + 2 * _t + 1] = 32 * _j + 16 + _t


def _mm2_body(x_ref, k1_ref, k2_ref, z1_ref, z2_ref):
    xb = x_ref[...]
    z1_ref[...] = jnp.dot(xb, k1_ref[...], preferred_element_type=jnp.float32)
    z2_ref[...] = jnp.dot(xb, k2_ref[...], preferred_element_type=jnp.float32)


def _mmadd_body(x_ref, s_ref, k0_ref, b_ref, o_ref):
    o_ref[...] = (jnp.dot(x_ref[...], k0_ref[...],
                          preferred_element_type=jnp.float32)
                  + s_ref[...] + b_ref[...][0:1, :])


def _sc_body(z1_hbm, z2_hbm, p0_hbm, p1_hbm, out_hbm,
             acc, ring0, ring1, ring2, ring3, col0, col1,
             row0, row1, gbuf0, gbuf1, sem0, sem1,
             semi0, semi1, semi2, semi3):
    cid = lax.axis_index("c")
    sid = lax.axis_index("s")
    base = sid * RPT
    rings = (ring0, ring1, ring2, ring3)
    cols = (col0, col1)
    rows = (row0, row1)
    gbufs = (gbuf0, gbuf1)
    sems = (sem0, sem1)
    semis = (semi0, semi1, semi2, semi3)

    def batch_body(bi, _):
        b = cid * (B // 2) + bi
        bN = b * N

        def zloop(i, _):
            for j in range(U // 16):
                gbuf0[i, pl.ds(j * 16, 16)] = jnp.zeros((16,), jnp.float32)
            return 0
        lax.fori_loop(0, CHUNK, zloop, 0)
        for k in range(RPT // CHUNK):
            pltpu.sync_copy(gbuf0, acc.at[pl.ds(base + k * CHUNK, CHUNK)])
        rem = RPT % CHUNK
        if rem:
            pltpu.sync_copy(gbuf0.at[pl.ds(0, rem)],
                            acc.at[pl.ds(base + RPT - rem, rem)])

        @pl.when(sid == NUM_TECS - 1)
        def _zero_tail():
            pltpu.sync_copy(gbuf0.at[pl.ds(0, TAIL)],
                            acc.at[pl.ds(RPT * NUM_TECS, TAIL)])
        plsc.subcore_barrier()

        for z_hbm, p_hbm in ((z1_hbm, p0_hbm), (z2_hbm, p1_hbm)):

            def fetch(k, q, p_hbm=p_hbm):
                pltpu.async_copy(p_hbm.at[sid, k], rings[q], semis[q])

            def fetch_wait(k, q, p_hbm=p_hbm):
                pltpu.make_async_copy(
                    p_hbm.at[sid, k], rings[q], semis[q]).wait()

            def stage(k, q, p, wait, bN=bN, z_hbm=z_hbm):
                if wait:
                    fetch_wait(k, q)
                rg = rings[q]
                cb = cols[p]
                for j in range(CHUNK // 16):
                    cb[pl.ds(j * 16, 16)] = rg[0, pl.ds(j * 16, 16)] + bN
                pltpu.async_copy(z_hbm.at[cb], gbufs[p], sems[p])

            def process(k, q, p, z_hbm=z_hbm):
                pltpu.make_async_copy(
                    z_hbm.at[cols[p]], gbufs[p], sems[p]).wait()
                rg = rings[q]
                gb = gbufs[p]

                def srow(t, _):
                    valv = (rg[2, pl.ds(t * 16, 16)].astype(jnp.float32)
                            * (1.0 / VSCALE))
                    for i in range(16):
                        row = t * 16 + i
                        v = valv[i]
                        for j in range(U // 16):
                            gb[row, pl.ds(j * 16, 16)] = (
                                gb[row, pl.ds(j * 16, 16)] * v)
                    return 0
                lax.fori_loop(0, CHUNK // 16, srow, 0)
                rb = rows[p]
                for j in range(CHUNK // 16):
                    rb[pl.ds(j * 16, 16)] = rg[1, pl.ds(j * 16, 16)]
                pltpu.sync_copy(gb, acc.at[rb], add=True)

            for q in range(4):
                fetch(q, q)
            stage(0, 0, 0, True)

            def quad_body(t, _):
                k = t * 4
                stage(k + 1, 1, 1, True)
                process(k, 0, 0)
                fetch(k + 4, 0)
                stage(k + 2, 2, 0, True)
                process(k + 1, 1, 1)
                fetch(k + 5, 1)
                stage(k + 3, 3, 1, True)
                process(k + 2, 2, 0)
                fetch(k + 6, 2)
                stage(k + 4, 0, 0, True)
                process(k + 3, 3, 1)
                fetch(k + 7, 3)
                return 0
            lax.fori_loop(0, NCHUNK // 4, quad_body, 0)
            pltpu.make_async_copy(
                z_hbm.at[cols[0]], gbufs[0], sems[0]).wait()
            for d in (1, 2, 3):
                fetch_wait(NCHUNK + d, (NCHUNK + d) % 4)
        plsc.subcore_barrier()
        pltpu.sync_copy(acc.at[pl.ds(base, RPT)],
                        out_hbm.at[pl.ds(bN + base, RPT)])

        @pl.when(sid == NUM_TECS - 1)
        def _flush_tail():
            pltpu.sync_copy(acc.at[pl.ds(RPT * NUM_TECS, TAIL)],
                            out_hbm.at[pl.ds(bN + RPT * NUM_TECS, TAIL)])
        return 0

    lax.fori_loop(0, B // 2, batch_body, 0)


_sc_spmm = functools.partial(
    pl.kernel,
    out_type=jax.ShapeDtypeStruct((M, U), jnp.float32),
    mesh=plsc.VectorSubcoreMesh(core_axis_name="c", subcore_axis_name="s"),
    scratch_types=[
        pltpu.VMEM_SHARED((N, U), jnp.float32),
        pltpu.VMEM((3, CHUNK), jnp.int32),
        pltpu.VMEM((3, CHUNK), jnp.int32),
        pltpu.VMEM((3, CHUNK), jnp.int32),
        pltpu.VMEM((3, CHUNK), jnp.int32),
        pltpu.VMEM((CHUNK,), jnp.int32),
        pltpu.VMEM((CHUNK,), jnp.int32),
        pltpu.VMEM((CHUNK,), jnp.int32),
        pltpu.VMEM((CHUNK,), jnp.int32),
        pltpu.VMEM((CHUNK, U), jnp.float32),
        pltpu.VMEM((CHUNK, U), jnp.float32),
        pltpu.SemaphoreType.DMA,
        pltpu.SemaphoreType.DMA,
        pltpu.SemaphoreType.DMA,
        pltpu.SemaphoreType.DMA,
        pltpu.SemaphoreType.DMA,
        pltpu.SemaphoreType.DMA,
    ],
)(_sc_body)


def _pack_edges(edge_index, values):
    pad = NUM_TECS * NCHUNK * CHUNK - E
    cols = jnp.pad(edge_index[1], (0, pad))
    rows = jnp.pad(edge_index[0], (0, pad))
    vals = jnp.pad(jnp.round(values * VSCALE).astype(jnp.int32), (0, pad))
    packed = jnp.stack([cols, rows, vals], 0)
    packed = packed.reshape(3, NUM_TECS, NCHUNK, CHUNK)
    packed = jnp.pad(packed, ((0, 0), (0, 0), (0, NCF - NCHUNK), (0, 0)))
    return jnp.transpose(packed, (1, 2, 0, 3))


def kernel(x, edge_index0, values0, edge_index1, values1, kernel, bias):
    xf = x.reshape(M, D)
    kw = kernel.reshape(D, 3, U)
    k0 = kw[:, 0, :]
    k1p = kw[:, 1, :]
    k2p = kw[:, 2, :]

    z1, z2 = pl.pallas_call(
        _mm2_body,
        grid=(M // _BM,),
        in_specs=[
            pl.BlockSpec((_BM, D), lambda i: (i, 0)),
            pl.BlockSpec((D, U), lambda i: (0, 0)),
            pl.BlockSpec((D, U), lambda i: (0, 0)),
        ],
        out_specs=[
            pl.BlockSpec((_BM, U), lambda i: (i, 0)),
            pl.BlockSpec((_BM, U), lambda i: (i, 0)),
        ],
        out_shape=[
            jax.ShapeDtypeStruct((M, U), jnp.float32),
            jax.ShapeDtypeStruct((M, U), jnp.float32),
        ],
    )(xf, k1p, k2p)

    s = _sc_spmm(z1, z2,
                 _pack_edges(edge_index0, values0),
                 _pack_edges(edge_index1, values1))

    bias2 = jnp.broadcast_to(bias, (8, U))
    out = pl.pallas_call(
        _mmadd_body,
        grid=(M // _BM,),
        in_specs=[
            pl.BlockSpec((_BM, D), lambda i: (i, 0)),
            pl.BlockSpec((_BM, U), lambda i: (i, 0)),
            pl.BlockSpec((D, U), lambda i: (0, 0)),
            pl.BlockSpec((8, U), lambda i: (0, 0)),
        ],
        out_specs=pl.BlockSpec((_BM, U), lambda i: (i, 0)),
        out_shape=jax.ShapeDtypeStruct((M, U), jnp.float32),
    )(xf, s, k0, bias2)

    return out.reshape(B, N, U)


# speedup vs baseline: 1.3264x; 1.3264x over previous
import functools

import jax
import jax.numpy as jnp
from jax import lax
from jax.experimental import pallas as pl
from jax.experimental.pallas import tpu as pltpu
from jax.experimental.pallas import tpu_sc as plsc

B = 8
N = 10000
D = 128
U = 128
E = 320000
M = B * N

NUM_TECS = 16
CHUNK = 128
NCHUNK = 158
EPT = NCHUNK * CHUNK
EPAD = NUM_TECS * EPT
RPT = 624
TAIL = N - RPT * NUM_TECS

_BM = 2000


def _mm2_body(x_ref, k1_ref, k2_ref, z1_ref, z2_ref):
    xb = x_ref[...]
    z1_ref[...] = jnp.dot(xb, k1_ref[...], preferred_element_type=jnp.float32)
    z2_ref[...] = jnp.dot(xb, k2_ref[...], preferred_element_type=jnp.float32)


def _mmadd_body(x_ref, s_ref, k0_ref, b_ref, o_ref):
    o_ref[...] = (jnp.dot(x_ref[...], k0_ref[...],
                          preferred_element_type=jnp.float32)
                  + s_ref[...] + b_ref[...][0:1, :])


def _sc_body(z1_hbm, z2_hbm, p0_hbm, v0_hbm, p1_hbm, v1_hbm, out_hbm,
             acc, ring_a, ring_b, vring_a, vring_b, cola, colb,
             gbuf_a, gbuf_b, semg_a, semg_b):
    cid = lax.axis_index("c")
    sid = lax.axis_index("s")
    base = sid * RPT
    rings = (ring_a, ring_b)
    vrings = (vring_a, vring_b)
    colbufs = (cola, colb)
    gbufs = (gbuf_a, gbuf_b)
    sems = (semg_a, semg_b)

    def batch_body(bi, _):
        b = cid * (B // 2) + bi
        bN = b * N

        def zloop(i, _):
            for j in range(U // 16):
                gbuf_a[i, pl.ds(j * 16, 16)] = jnp.zeros((16,), jnp.float32)
            return 0
        lax.fori_loop(0, CHUNK, zloop, 0)
        for k in range(RPT // CHUNK):
            pltpu.sync_copy(gbuf_a, acc.at[pl.ds(base + k * CHUNK, CHUNK)])
        rem = RPT % CHUNK
        if rem:
            pltpu.sync_copy(gbuf_a.at[pl.ds(0, rem)],
                            acc.at[pl.ds(base + RPT - rem, rem)])

        @pl.when(sid == NUM_TECS - 1)
        def _zero_tail():
            pltpu.sync_copy(gbuf_a.at[pl.ds(0, TAIL)],
                            acc.at[pl.ds(RPT * NUM_TECS, TAIL)])
        plsc.subcore_barrier()

        for z_hbm, p_hbm, v_hbm in ((z1_hbm, p0_hbm, v0_hbm),
                                    (z2_hbm, p1_hbm, v1_hbm)):

            def stage_and_gather(k, p, z_hbm=z_hbm, p_hbm=p_hbm,
                                 v_hbm=v_hbm, bN=bN):
                rg = rings[p]
                cb = colbufs[p]
                pltpu.sync_copy(p_hbm.at[sid, k], rg)
                pltpu.sync_copy(v_hbm.at[sid, k], vrings[p])
                for j in range(CHUNK // 16):
                    cb[pl.ds(j * 16, 16)] = rg[0, pl.ds(j * 16, 16)] + bN
                pltpu.async_copy(z_hbm.at[cb], gbufs[p], sems[p])

            def process(k, p, z_hbm=z_hbm):
                pltpu.make_async_copy(
                    z_hbm.at[colbufs[p]], gbufs[p], sems[p]).wait()
                rg = rings[p]
                gb = gbufs[p]

                def srow(t, _):
                    valv = vrings[p][pl.ds(t * 16, 16)]
                    for i in range(16):
                        r = t * 16 + i
                        v = valv[i]
                        for j in range(U // 16):
                            gb[r, pl.ds(j * 16, 16)] = (
                                gb[r, pl.ds(j * 16, 16)] * v)
                    return 0
                lax.fori_loop(0, CHUNK // 16, srow, 0)
                pltpu.sync_copy(gb, acc.at[rg.at[1]], add=True)

            stage_and_gather(0, 0)

            def pair_body(k2, _):
                k = k2 * 2
                stage_and_gather(k + 1, 1)
                process(k, 0)
                stage_and_gather(k + 2, 0)
                process(k + 1, 1)
                return 0
            lax.fori_loop(0, NCHUNK // 2 - 1, pair_body, 0)
            stage_and_gather(NCHUNK - 1, 1)
            process(NCHUNK - 2, 0)
            process(NCHUNK - 1, 1)
        plsc.subcore_barrier()
        pltpu.sync_copy(acc.at[pl.ds(base, RPT)],
                        out_hbm.at[pl.ds(bN + base, RPT)])

        @pl.when(sid == NUM_TECS - 1)
        def _flush_tail():
            pltpu.sync_copy(acc.at[pl.ds(RPT * NUM_TECS, TAIL)],
                            out_hbm.at[pl.ds(bN + RPT * NUM_TECS, TAIL)])
        return 0

    lax.fori_loop(0, B // 2, batch_body, 0)


_sc_spmm = functools.partial(
    pl.kernel,
    out_type=jax.ShapeDtypeStruct((M, U), jnp.float32),
    mesh=plsc.VectorSubcoreMesh(core_axis_name="c", subcore_axis_name="s"),
    scratch_types=[
        pltpu.VMEM_SHARED((N, U), jnp.float32),
        pltpu.VMEM((2, CHUNK), jnp.int32),
        pltpu.VMEM((2, CHUNK), jnp.int32),
        pltpu.VMEM((CHUNK,), jnp.float32),
        pltpu.VMEM((CHUNK,), jnp.float32),
        pltpu.VMEM((CHUNK,), jnp.int32),
        pltpu.VMEM((CHUNK,), jnp.int32),
        pltpu.VMEM((CHUNK, U), jnp.float32),
        pltpu.VMEM((CHUNK, U), jnp.float32),
        pltpu.SemaphoreType.DMA,
        pltpu.SemaphoreType.DMA,
    ],
)(_sc_body)


def _pack_edges(edge_index, values):
    pad = EPAD - E
    cols = jnp.pad(edge_index[1], (0, pad))
    rows = jnp.pad(edge_index[0], (0, pad))
    vals = jnp.pad(values, (0, pad))
    packed = jnp.stack([cols, rows], 0).reshape(2, NUM_TECS, NCHUNK, CHUNK)
    return (jnp.transpose(packed, (1, 2, 0, 3)),
            vals.reshape(NUM_TECS, NCHUNK, CHUNK))


def kernel(x, edge_index0, values0, edge_index1, values1, kernel, bias):
    xf = x.reshape(M, D)
    kw = kernel.reshape(D, 3, U)
    k0, k1, k2 = kw[:, 0, :], kw[:, 1, :], kw[:, 2, :]

    z1, z2 = pl.pallas_call(
        _mm2_body,
        grid=(M // _BM,),
        in_specs=[
            pl.BlockSpec((_BM, D), lambda i: (i, 0)),
            pl.BlockSpec((D, U), lambda i: (0, 0)),
            pl.BlockSpec((D, U), lambda i: (0, 0)),
        ],
        out_specs=[
            pl.BlockSpec((_BM, U), lambda i: (i, 0)),
            pl.BlockSpec((_BM, U), lambda i: (i, 0)),
        ],
        out_shape=[
            jax.ShapeDtypeStruct((M, U), jnp.float32),
            jax.ShapeDtypeStruct((M, U), jnp.float32),
        ],
    )(xf, k1, k2)

    p0, v0 = _pack_edges(edge_index0, values0)
    p1, v1 = _pack_edges(edge_index1, values1)
    s = _sc_spmm(z1, z2, p0, v0, p1, v1)

    bias2 = jnp.broadcast_to(bias, (8, U))
    out = pl.pallas_call(
        _mmadd_body,
        grid=(M // _BM,),
        in_specs=[
            pl.BlockSpec((_BM, D), lambda i: (i, 0)),
            pl.BlockSpec((_BM, U), lambda i: (i, 0)),
            pl.BlockSpec((D, U), lambda i: (0, 0)),
            pl.BlockSpec((8, U), lambda i: (0, 0)),
        ],
        out_specs=pl.BlockSpec((_BM, U), lambda i: (i, 0)),
        out_shape=jax.ShapeDtypeStruct((M, U), jnp.float32),
    )(xf, s, k0, bias2)

    return out.reshape(B, N, U)


# speedup vs baseline: 1.4638x; 1.1036x over previous
import functools

import jax
import jax.numpy as jnp
from jax import lax
from jax.experimental import pallas as pl
from jax.experimental.pallas import tpu as pltpu
from jax.experimental.pallas import tpu_sc as plsc

B = 8
N = 10000
D = 128
U = 128
E = 320000
M = B * N

NUM_TECS = 16
CHUNK = 128
NCHUNK = 158
EPT = NCHUNK * CHUNK
EPAD = NUM_TECS * EPT
RPT = 624
VSCALE = float(2 ** 24)
TAIL = N - RPT * NUM_TECS

_BM = 2000


def _mm2_body(x_ref, k1_ref, k2_ref, z1_ref, z2_ref):
    xb = x_ref[...]
    z1_ref[...] = jnp.dot(xb, k1_ref[...], preferred_element_type=jnp.float32)
    z2_ref[...] = jnp.dot(xb, k2_ref[...], preferred_element_type=jnp.float32)


def _mmadd_body(x_ref, s_ref, k0_ref, b_ref, o_ref):
    o_ref[...] = (jnp.dot(x_ref[...], k0_ref[...],
                          preferred_element_type=jnp.float32)
                  + s_ref[...] + b_ref[...][0:1, :])


def _sc_body(z1_hbm, z2_hbm, p0_hbm, p1_hbm, out_hbm,
             acc, ring_a, ring_b, cola, colb,
             gbuf_a, gbuf_b, semg_a, semg_b):
    cid = lax.axis_index("c")
    sid = lax.axis_index("s")
    base = sid * RPT
    rings = (ring_a, ring_b)
    colbufs = (cola, colb)
    gbufs = (gbuf_a, gbuf_b)
    sems = (semg_a, semg_b)

    def batch_body(bi, _):
        b = cid * (B // 2) + bi
        bN = b * N

        def zloop(i, _):
            for j in range(U // 16):
                gbuf_a[i, pl.ds(j * 16, 16)] = jnp.zeros((16,), jnp.float32)
            return 0
        lax.fori_loop(0, CHUNK, zloop, 0)
        for k in range(RPT // CHUNK):
            pltpu.sync_copy(gbuf_a, acc.at[pl.ds(base + k * CHUNK, CHUNK)])
        rem = RPT % CHUNK
        if rem:
            pltpu.sync_copy(gbuf_a.at[pl.ds(0, rem)],
                            acc.at[pl.ds(base + RPT - rem, rem)])

        @pl.when(sid == NUM_TECS - 1)
        def _zero_tail():
            pltpu.sync_copy(gbuf_a.at[pl.ds(0, TAIL)],
                            acc.at[pl.ds(RPT * NUM_TECS, TAIL)])
        plsc.subcore_barrier()

        for z_hbm, p_hbm in ((z1_hbm, p0_hbm), (z2_hbm, p1_hbm)):

            def stage_and_gather(k, p, z_hbm=z_hbm, p_hbm=p_hbm, bN=bN):
                rg = rings[p]
                cb = colbufs[p]
                pltpu.sync_copy(p_hbm.at[sid, k], rg)
                for j in range(CHUNK // 16):
                    cb[pl.ds(j * 16, 16)] = rg[0, pl.ds(j * 16, 16)] + bN
                pltpu.async_copy(z_hbm.at[cb], gbufs[p], sems[p])

            def process(k, p, z_hbm=z_hbm):
                pltpu.make_async_copy(
                    z_hbm.at[colbufs[p]], gbufs[p], sems[p]).wait()
                rg = rings[p]
                gb = gbufs[p]

                def srow(t, _):
                    valv = (rg[2, pl.ds(t * 16, 16)].astype(jnp.float32)
                            * (1.0 / VSCALE))
                    for i in range(16):
                        r = t * 16 + i
                        v = valv[i]
                        for j in range(U // 16):
                            gb[r, pl.ds(j * 16, 16)] = (
                                gb[r, pl.ds(j * 16, 16)] * v)
                    return 0
                lax.fori_loop(0, CHUNK // 16, srow, 0)
                pltpu.sync_copy(gb, acc.at[rg.at[1]], add=True)

            stage_and_gather(0, 0)

            def pair_body(k2, _):
                k = k2 * 2
                stage_and_gather(k + 1, 1)
                process(k, 0)
                stage_and_gather(k + 2, 0)
                process(k + 1, 1)
                return 0
            lax.fori_loop(0, NCHUNK // 2 - 1, pair_body, 0)
            stage_and_gather(NCHUNK - 1, 1)
            process(NCHUNK - 2, 0)
            process(NCHUNK - 1, 1)
        plsc.subcore_barrier()
        pltpu.sync_copy(acc.at[pl.ds(base, RPT)],
                        out_hbm.at[pl.ds(bN + base, RPT)])

        @pl.when(sid == NUM_TECS - 1)
        def _flush_tail():
            pltpu.sync_copy(acc.at[pl.ds(RPT * NUM_TECS, TAIL)],
                            out_hbm.at[pl.ds(bN + RPT * NUM_TECS, TAIL)])
        return 0

    lax.fori_loop(0, B // 2, batch_body, 0)


_sc_spmm = functools.partial(
    pl.kernel,
    out_type=jax.ShapeDtypeStruct((M, U), jnp.float32),
    mesh=plsc.VectorSubcoreMesh(core_axis_name="c", subcore_axis_name="s"),
    scratch_types=[
        pltpu.VMEM_SHARED((N, U), jnp.float32),
        pltpu.VMEM((4, CHUNK), jnp.int32),
        pltpu.VMEM((4, CHUNK), jnp.int32),
        pltpu.VMEM((CHUNK,), jnp.int32),
        pltpu.VMEM((CHUNK,), jnp.int32),
        pltpu.VMEM((CHUNK, U), jnp.float32),
        pltpu.VMEM((CHUNK, U), jnp.float32),
        pltpu.SemaphoreType.DMA,
        pltpu.SemaphoreType.DMA,
    ],
)(_sc_body)


def _pack_edges(edge_index, values):
    pad = EPAD - E
    cols = jnp.pad(edge_index[1], (0, pad))
    rows = jnp.pad(edge_index[0], (0, pad))
    vals = jnp.pad(jnp.round(values * VSCALE).astype(jnp.int32), (0, pad))
    zero = jnp.zeros_like(cols)
    packed = jnp.stack([cols, rows, vals, zero], 0)
    packed = packed.reshape(4, NUM_TECS, NCHUNK, CHUNK)
    return jnp.transpose(packed, (1, 2, 0, 3))


def kernel(x, edge_index0, values0, edge_index1, values1, kernel, bias):
    xf = x.reshape(M, D)
    kw = kernel.reshape(D, 3, U)
    k0, k1, k2 = kw[:, 0, :], kw[:, 1, :], kw[:, 2, :]

    z1, z2 = pl.pallas_call(
        _mm2_body,
        grid=(M // _BM,),
        in_specs=[
            pl.BlockSpec((_BM, D), lambda i: (i, 0)),
            pl.BlockSpec((D, U), lambda i: (0, 0)),
            pl.BlockSpec((D, U), lambda i: (0, 0)),
        ],
        out_specs=[
            pl.BlockSpec((_BM, U), lambda i: (i, 0)),
            pl.BlockSpec((_BM, U), lambda i: (i, 0)),
        ],
        out_shape=[
            jax.ShapeDtypeStruct((M, U), jnp.float32),
            jax.ShapeDtypeStruct((M, U), jnp.float32),
        ],
    )(xf, k1, k2)

    s = _sc_spmm(z1, z2,
                 _pack_edges(edge_index0, values0),
                 _pack_edges(edge_index1, values1))

    bias2 = jnp.broadcast_to(bias, (8, U))
    out = pl.pallas_call(
        _mmadd_body,
        grid=(M // _BM,),
        in_specs=[
            pl.BlockSpec((_BM, D), lambda i: (i, 0)),
            pl.BlockSpec((_BM, U), lambda i: (i, 0)),
            pl.BlockSpec((D, U), lambda i: (0, 0)),
            pl.BlockSpec((8, U), lambda i: (0, 0)),
        ],
        out_specs=pl.BlockSpec((_BM, U), lambda i: (i, 0)),
        out_shape=jax.ShapeDtypeStruct((M, U), jnp.float32),
    )(xf, s, k0, bias2)

    return out.reshape(B, N, U)
